# baseline (speedup 1.0000x reference)
import functools

import jax
import jax.numpy as jnp
from jax import lax
from jax.experimental import pallas as pl
from jax.experimental.pallas import tpu as pltpu
from jax.experimental.pallas import tpu_sc as plsc


def _sc_gather(xs, weight):
    B = xs.shape[0]
    V, D = weight.shape
    info = plsc.get_sparse_core_info()
    nc, ns = info.num_cores, info.num_subcores
    nw = nc * ns
    b_per_w = B // nw

    mesh = plsc.VectorSubcoreMesh(core_axis_name="c", subcore_axis_name="s")

    @functools.partial(
        pl.kernel,
        mesh=mesh,
        out_type=jax.ShapeDtypeStruct((B, D), jnp.float32),
        scratch_types=[
            pltpu.VMEM((b_per_w,), jnp.int32),
            pltpu.VMEM((b_per_w, D), jnp.float32),
            pltpu.SemaphoreType.DMA,
        ],
    )
    def gather_kernel(xs_hbm, w_hbm, out_hbm, idx_v, rows_v, sem):
        wid = lax.axis_index("s") * nc + lax.axis_index("c")
        base = wid * b_per_w
        pltpu.sync_copy(xs_hbm.at[pl.ds(base, b_per_w)], idx_v)
        pltpu.async_copy(w_hbm.at[idx_v], rows_v, sem).wait()
        pltpu.sync_copy(rows_v, out_hbm.at[pl.ds(base, b_per_w)])

    return gather_kernel(xs, weight)


def _tc_project(emb, weight, vocab_block=1024):
    B, D = emb.shape
    V = weight.shape[0]
    grid = pl.cdiv(V, vocab_block)

    def body(emb_ref, w_ref, out_ref):
        a = emb_ref[...].astype(jnp.bfloat16)
        b = w_ref[...].astype(jnp.bfloat16)
        out_ref[...] = lax.dot_general(
            a, b, (((1,), (1,)), ((), ())),
            preferred_element_type=jnp.float32,
        )

    return pl.pallas_call(
        body,
        grid=(grid,),
        in_specs=[
            pl.BlockSpec((B, D), lambda i: (0, 0)),
            pl.BlockSpec((vocab_block, D), lambda i: (i, 0)),
        ],
        out_specs=pl.BlockSpec((B, vocab_block), lambda i: (0, i)),
        out_shape=jax.ShapeDtypeStruct((B, V), jnp.float32),
    )(emb, weight)


def kernel(xs, weight):
    emb = _sc_gather(xs.astype(jnp.int32), weight)
    return _tc_project(emb, weight)
